# baseline (speedup 1.0000x reference)
import jax
import jax.numpy as jnp
from jax import lax
from jax.experimental import pallas as pl
from jax.experimental.pallas import tpu as pltpu, tpu_sc as plsc

_EMBED_DIM = 64


def _row_gather_body(table_hbm, idx_hbm, out_hbm, idx_s):
    @pl.when(lax.axis_index("c") == 0)
    def _():
        pltpu.sync_copy(idx_hbm, idx_s)
        i = idx_s[0]
        pltpu.sync_copy(table_hbm.at[pl.ds(i, 1), :], out_hbm)


def kernel(table, domain_id):
    idx = jnp.asarray(domain_id, jnp.int32).reshape((1,))
    mesh = plsc.ScalarSubcoreMesh(axis_name="c", num_cores=2)
    gather = pl.kernel(
        _row_gather_body,
        mesh=mesh,
        out_type=jax.ShapeDtypeStruct((1, _EMBED_DIM), jnp.float32),
        scratch_types=[
            pltpu.SMEM((1,), jnp.int32),
        ],
        compiler_params=pltpu.CompilerParams(skip_device_barrier=True),
    )
    out = gather(table, idx)
    return out.reshape((_EMBED_DIM,))


# speedup vs baseline: 17.0499x; 17.0499x over previous
import jax
import jax.numpy as jnp
from jax import lax
from jax.experimental import pallas as pl
from jax.experimental.pallas import tpu as pltpu, tpu_sc as plsc

_EMBED_DIM = 64
_LANES = 16
_WIN = 128


def _col_gather_body(tableT_hbm, idx_hbm, out_hbm, idx_v, win_v, out_v):
    cid = lax.axis_index("c")
    sid = lax.axis_index("s")

    @pl.when(jnp.logical_and(cid == 0, sid == 0))
    def _():
        pltpu.sync_copy(idx_hbm, idx_v)
        v = idx_v[...]
        i = lax.reduce_max(v, axes=(0,))
        base = pl.multiple_of((i // _WIN) * _WIN, _WIN)
        pltpu.sync_copy(tableT_hbm.at[:, pl.ds(base, _WIN)], win_v)
        col = v % _WIN
        for g in range(_EMBED_DIM // _LANES):
            rows = lax.iota(jnp.int32, _LANES) + g * _LANES
            vals = plsc.load_gather(win_v, [rows, col])
            out_v[pl.ds(g * _LANES, _LANES)] = vals
        pltpu.sync_copy(out_v, out_hbm)


def kernel(table, domain_id):
    idx = jnp.full((_LANES,), domain_id, dtype=jnp.int32)
    tableT = table.T
    mesh = plsc.VectorSubcoreMesh(core_axis_name="c", subcore_axis_name="s")
    gather = pl.kernel(
        _col_gather_body,
        mesh=mesh,
        out_type=jax.ShapeDtypeStruct((_EMBED_DIM,), jnp.float32),
        scratch_types=[
            pltpu.VMEM((_LANES,), jnp.int32),
            pltpu.VMEM((_EMBED_DIM, _WIN), jnp.float32),
            pltpu.VMEM((_EMBED_DIM,), jnp.float32),
        ],
        compiler_params=pltpu.CompilerParams(
            skip_device_barrier=True,
            disable_bounds_checks=True,
            needs_layout_passes=False,
        ),
    )
    return gather(tableT, idx)


# speedup vs baseline: 18.7657x; 1.1006x over previous
import jax
import jax.numpy as jnp
from jax import lax
from jax.experimental import pallas as pl
from jax.experimental.pallas import tpu as pltpu, tpu_sc as plsc

_EMBED_DIM = 64
_LANES = 16
_WIN = 128


def _col_gather_body(tableT_hbm, idx_hbm, out_hbm, idx_v, win_v, out_v):
    cid = lax.axis_index("c")
    sid = lax.axis_index("s")

    @pl.when(jnp.logical_and(cid == 0, sid == 0))
    def _():
        pltpu.sync_copy(idx_hbm, idx_v)
        v = idx_v[...]
        i = lax.reduce_max(v, axes=(0,))
        base = pl.multiple_of((i // _WIN) * _WIN, _WIN)
        pltpu.sync_copy(tableT_hbm.at[:, pl.ds(base, _WIN)], win_v)
        col = v % _WIN
        for g in range(_EMBED_DIM // _LANES):
            rows = lax.iota(jnp.int32, _LANES) + g * _LANES
            vals = plsc.load_gather(win_v, [rows, col])
            out_v[pl.ds(g * _LANES, _LANES)] = vals
        pltpu.sync_copy(out_v, out_hbm)


def kernel(table, domain_id):
    idx = jnp.full((_LANES,), domain_id, dtype=jnp.int32)
    tableT = table.T
    mesh = plsc.VectorSubcoreMesh(
        core_axis_name="c", subcore_axis_name="s", num_cores=1
    )
    gather = pl.kernel(
        _col_gather_body,
        mesh=mesh,
        out_type=jax.ShapeDtypeStruct((_EMBED_DIM,), jnp.float32),
        scratch_types=[
            pltpu.VMEM((_LANES,), jnp.int32),
            pltpu.VMEM((_EMBED_DIM, _WIN), jnp.float32),
            pltpu.VMEM((_EMBED_DIM,), jnp.float32),
        ],
        compiler_params=pltpu.CompilerParams(
            skip_device_barrier=True,
            disable_bounds_checks=True,
            needs_layout_passes=False,
        ),
    )
    return gather(tableT, idx)


# speedup vs baseline: 18.9404x; 1.0093x over previous
import jax
import jax.numpy as jnp
from jax import lax
from jax.experimental import pallas as pl
from jax.experimental.pallas import tpu as pltpu, tpu_sc as plsc

_EMBED_DIM = 64
_LANES = 16
_WIN = 128


def _col_gather_body(tableT_hbm, idx_hbm, out_hbm, idx_v, win_v, out_v):
    cid = lax.axis_index("c")
    sid = lax.axis_index("s")

    @pl.when(jnp.logical_and(cid == 0, sid == 0))
    def _():
        pltpu.sync_copy(idx_hbm, idx_v)
        v = idx_v[...]
        i = lax.reduce_max(v, axes=(0,))
        base = pl.multiple_of((i // _WIN) * _WIN, _WIN)
        pltpu.sync_copy(tableT_hbm.at[:, pl.ds(base, _WIN)], win_v)
        col = v % _WIN
        for g in range(_EMBED_DIM // _LANES):
            rows = lax.iota(jnp.int32, _LANES) + g * _LANES
            vals = plsc.load_gather(win_v, [rows, col])
            out_v[pl.ds(g * _LANES, _LANES)] = vals
        pltpu.sync_copy(out_v, out_hbm)


def kernel(table, domain_id):
    idx = jnp.full((_LANES,), domain_id, dtype=jnp.int32)
    tableT = table.T
    mesh = plsc.VectorSubcoreMesh(
        core_axis_name="c", subcore_axis_name="s", num_cores=1, num_subcores=1
    )
    gather = pl.kernel(
        _col_gather_body,
        mesh=mesh,
        out_type=jax.ShapeDtypeStruct((_EMBED_DIM,), jnp.float32),
        scratch_types=[
            pltpu.VMEM((_LANES,), jnp.int32),
            pltpu.VMEM((_EMBED_DIM, _WIN), jnp.float32),
            pltpu.VMEM((_EMBED_DIM,), jnp.float32),
        ],
        compiler_params=pltpu.CompilerParams(
            skip_device_barrier=True,
            disable_bounds_checks=True,
            needs_layout_passes=False,
        ),
    )
    return gather(tableT, idx)
